# Initial kernel scaffold; baseline (speedup 1.0000x reference)
import functools

import jax
import jax.numpy as jnp
from jax import lax
from jax.experimental import pallas as pl
from jax.experimental.pallas import tpu as pltpu
from jax.experimental.pallas import tpu_sc as plsc

NN = 10000
NE = 10000
NNZ = 320000
C = 128
NC = 2
NS = 16
NW = NC * NS
EPW = NNZ // NW
CH = 80
NCH = EPW // CH
ACC_ROWS = 10240
ZR = 640
OR = NN // NS


def _phase_body(with_db, *refs):
    if with_db:
        (tab_hbm, gi_hbm, si_hbm, hew_hbm, acc_out, dp_out, bp_out,
         gi_v, si_v, rows_v, hew_v, d_v, b_v, acc_sh, sem) = refs
    else:
        (tab_hbm, gi_hbm, si_hbm, acc_out,
         gi_v, si_v, rows_v, acc_sh, sem) = refs

    cid = lax.axis_index("c")
    sid = lax.axis_index("s")
    wid = sid * NC + cid

    pltpu.sync_copy(gi_hbm.at[wid], gi_v)
    pltpu.sync_copy(si_hbm.at[wid], si_v)
    if with_db:
        pltpu.sync_copy(hew_hbm, hew_v)

    z16 = jnp.zeros((16,), jnp.float32)

    def zrow(i, carry):
        rows_v[i // 8, pl.ds((i % 8) * 16, 16)] = z16
        return carry

    lax.fori_loop(0, CH * 8, zrow, 0)
    for k in range(ZR // CH):
        pltpu.sync_copy(rows_v, acc_sh.at[pl.ds(sid * ZR + k * CH, CH)])

    if with_db:
        def zdb(i, carry):
            d_v[pl.ds(i * 16, 16)] = z16
            b_v[pl.ds(i * 16, 16)] = z16
            return carry
        lax.fori_loop(0, NN // 16, zdb, 0)

    plsc.subcore_barrier()

    ones16 = jnp.ones((16,), jnp.float32)

    def chunk(c, carry):
        pltpu.async_copy(tab_hbm.at[gi_v.at[c]], rows_v, sem).wait()
        pltpu.sync_copy(rows_v, acc_sh.at[si_v.at[c]], add=True)
        if with_db:
            for g in range(CH // 16):
                s16 = gi_v[c, pl.ds(g * 16, 16)]
                d16 = si_v[c, pl.ds(g * 16, 16)]
                w16 = plsc.load_gather(hew_v, [d16])
                plsc.addupdate_scatter(d_v, [s16], w16)
                plsc.addupdate_scatter(b_v, [d16], ones16)
        return carry

    lax.fori_loop(0, NCH, chunk, 0)
    plsc.subcore_barrier()

    pltpu.sync_copy(acc_sh.at[pl.ds(sid * OR, OR)],
                    acc_out.at[cid, pl.ds(sid * OR, OR)])
    if with_db:
        pltpu.sync_copy(d_v, dp_out.at[wid])
        pltpu.sync_copy(b_v, bp_out.at[wid])


_MESH = plsc.VectorSubcoreMesh(core_axis_name="c", subcore_axis_name="s",
                               num_cores=NC, num_subcores=NS)

_phase1 = pl.kernel(
    functools.partial(_phase_body, True),
    out_type=(
        jax.ShapeDtypeStruct((NC, NN, C), jnp.float32),
        jax.ShapeDtypeStruct((NW, NN), jnp.float32),
        jax.ShapeDtypeStruct((NW, NE), jnp.float32),
    ),
    mesh=_MESH,
    scratch_types=(
        pltpu.VMEM((NCH, CH), jnp.int32),
        pltpu.VMEM((NCH, CH), jnp.int32),
        pltpu.VMEM((CH, C), jnp.float32),
        pltpu.VMEM((NE,), jnp.float32),
        pltpu.VMEM((NN,), jnp.float32),
        pltpu.VMEM((NE,), jnp.float32),
        pltpu.VMEM_SHARED((ACC_ROWS, C), jnp.float32),
        pltpu.SemaphoreType.DMA,
    ),
)

_phase2 = pl.kernel(
    functools.partial(_phase_body, False),
    out_type=jax.ShapeDtypeStruct((NC, NN, C), jnp.float32),
    mesh=_MESH,
    scratch_types=(
        pltpu.VMEM((NCH, CH), jnp.int32),
        pltpu.VMEM((NCH, CH), jnp.int32),
        pltpu.VMEM((CH, C), jnp.float32),
        pltpu.VMEM_SHARED((ACC_ROWS, C), jnp.float32),
        pltpu.SemaphoreType.DMA,
    ),
)


def _combine_edges_body(ep_ref, bp_ref, out_ref):
    bsum = jnp.sum(bp_ref[...], axis=0)
    binv = jnp.where(bsum > 0, 1.0 / bsum, 0.0)
    out_ref[...] = (ep_ref[0] + ep_ref[1]) * binv[:, None]


def _finish_body(np_ref, dp_ref, w_ref, b_ref, out_ref):
    dsum = jnp.sum(dp_ref[...], axis=0)
    dinv = jnp.where(dsum > 0, 1.0 / dsum, 0.0)
    t = (np_ref[0] + np_ref[1]) * dinv[:, None]
    out_ref[...] = (jnp.dot(t, w_ref[...], preferred_element_type=jnp.float32)
                    + b_ref[...])


def kernel(x, HE, HEW, W, b):
    src = HE[0].reshape(NW, NCH, CH)
    dst = HE[1].reshape(NW, NCH, CH)

    ep, dp, bp = _phase1(x, src, dst, HEW)

    ef = pl.pallas_call(
        _combine_edges_body,
        out_shape=jax.ShapeDtypeStruct((NE, C), jnp.float32),
    )(ep, bp)

    npar = _phase2(ef, dst, src)

    out = pl.pallas_call(
        _finish_body,
        out_shape=jax.ShapeDtypeStruct((NN, C), jnp.float32),
    )(npar, dp, W, b.reshape(1, C))
    return out


# speedup vs baseline: 15.3573x; 15.3573x over previous
import functools

import jax
import jax.numpy as jnp
from jax import lax
from jax.experimental import pallas as pl
from jax.experimental.pallas import tpu as pltpu
from jax.experimental.pallas import tpu_sc as plsc

NN = 10000
NE = 10000
NNZ = 320000
C = 128
HC = C // 2
NC = 2
NS = 16
NW = NC * NS
CH = 80
EPT = NNZ // NS
NCHP = EPT // CH
EPW = NNZ // NW
NCHD = EPW // CH
ACC_ROWS = 10240
ZR = ACC_ROWS // NS


def _zero_vec_loop(ref, rows, cols):
    z16 = jnp.zeros((16,), jnp.float32)
    g = cols // 16

    def body(i, carry):
        ref[i // g, pl.ds((i % g) * 16, 16)] = z16
        return carry

    lax.fori_loop(0, rows * g, body, 0)


def _phase_body(tab_hbm, gi_hbm, si_hbm, acc_out,
                gi_v, si_v, rows_v, acc_sh, sem):
    cid = lax.axis_index("c")
    sid = lax.axis_index("s")

    pltpu.sync_copy(gi_hbm.at[sid], gi_v)
    pltpu.sync_copy(si_hbm.at[sid], si_v)

    _zero_vec_loop(rows_v, CH, HC)
    zbase = pl.multiple_of(sid * ZR, 8)
    for k in range(ZR // CH):
        pltpu.sync_copy(rows_v, acc_sh.at[pl.ds(zbase + k * CH, CH)])
    plsc.subcore_barrier()

    tab = tab_hbm.at[cid]

    def chunk(c, carry):
        pltpu.async_copy(tab.at[gi_v.at[c]], rows_v, sem).wait()
        pltpu.sync_copy(rows_v, acc_sh.at[si_v.at[c]], add=True)
        return carry

    lax.fori_loop(0, NCHP, chunk, 0)
    plsc.subcore_barrier()

    pltpu.sync_copy(acc_sh.at[pl.ds(zbase, ZR)],
                    acc_out.at[cid, pl.ds(zbase, ZR)])


def _degree_body(gi_hbm, si_hbm, hew_hbm, dp_out, bp_out,
                 gi_v, si_v, hew_v, d_v, b_v):
    cid = lax.axis_index("c")
    sid = lax.axis_index("s")
    wid = sid * NC + cid

    pltpu.sync_copy(gi_hbm.at[wid], gi_v)
    pltpu.sync_copy(si_hbm.at[wid], si_v)
    pltpu.sync_copy(hew_hbm, hew_v)

    z16 = jnp.zeros((16,), jnp.float32)

    def zdb(i, carry):
        d_v[pl.ds(i * 16, 16)] = z16
        b_v[pl.ds(i * 16, 16)] = z16
        return carry

    lax.fori_loop(0, NN // 16, zdb, 0)

    ones16 = jnp.ones((16,), jnp.float32)

    def chunk(c, carry):
        for g in range(CH // 16):
            s16 = gi_v[c, pl.ds(g * 16, 16)]
            d16 = si_v[c, pl.ds(g * 16, 16)]
            w16 = plsc.load_gather(hew_v, [d16])
            plsc.addupdate_scatter(d_v, [s16], w16)
            plsc.addupdate_scatter(b_v, [d16], ones16)
        return carry

    lax.fori_loop(0, NCHD, chunk, 0)

    obase = pl.multiple_of(wid * NN, 8)
    pltpu.sync_copy(d_v, dp_out.at[pl.ds(obase, NN)])
    pltpu.sync_copy(b_v, bp_out.at[pl.ds(obase, NN)])


_MESH = plsc.VectorSubcoreMesh(core_axis_name="c", subcore_axis_name="s",
                               num_cores=NC, num_subcores=NS)
_SC_PARAMS = pltpu.CompilerParams(needs_layout_passes=False,
                                  use_tc_tiling_on_sc=False)

_degree = pl.kernel(
    _degree_body,
    out_type=(
        jax.ShapeDtypeStruct((NW * NN,), jnp.float32),
        jax.ShapeDtypeStruct((NW * NE,), jnp.float32),
    ),
    mesh=_MESH,
    compiler_params=_SC_PARAMS,
    scratch_types=(
        pltpu.VMEM((NCHD, CH), jnp.int32),
        pltpu.VMEM((NCHD, CH), jnp.int32),
        pltpu.VMEM((NE,), jnp.float32),
        pltpu.VMEM((NN,), jnp.float32),
        pltpu.VMEM((NE,), jnp.float32),
    ),
)

_phase = pl.kernel(
    _phase_body,
    out_type=jax.ShapeDtypeStruct((NC, ACC_ROWS, HC), jnp.float32),
    mesh=_MESH,
    compiler_params=_SC_PARAMS,
    scratch_types=(
        pltpu.VMEM((NCHP, CH), jnp.int32),
        pltpu.VMEM((NCHP, CH), jnp.int32),
        pltpu.VMEM((CH, HC), jnp.float32),
        pltpu.VMEM_SHARED((ACC_ROWS, HC), jnp.float32),
        pltpu.SemaphoreType.DMA,
    ),
)


def _combine_edges_body(ep_ref, bp_ref, out_ref):
    bsum = jnp.sum(bp_ref[...], axis=0)
    binv = jnp.where(bsum > 0, 1.0 / bsum, 0.0)
    out_ref[0] = ep_ref[0, :NE] * binv[:, None]
    out_ref[1] = ep_ref[1, :NE] * binv[:, None]


def _finish_body(np_ref, dp_ref, w_ref, b_ref, out_ref):
    dsum = jnp.sum(dp_ref[...], axis=0)
    dinv = jnp.where(dsum > 0, 1.0 / dsum, 0.0)
    t = jnp.concatenate([np_ref[0, :NN], np_ref[1, :NN]], axis=1)
    t = t * dinv[:, None]
    out_ref[...] = (jnp.dot(t, w_ref[...], preferred_element_type=jnp.float32)
                    + b_ref[...])


def kernel(x, HE, HEW, W, b):
    src = HE[0]
    dst = HE[1]
    src_w = src.reshape(NW, NCHD, CH)
    dst_w = dst.reshape(NW, NCHD, CH)
    src_t = src.reshape(NS, NCHP, CH)
    dst_t = dst.reshape(NS, NCHP, CH)
    xs = x.reshape(NN, NC, HC).transpose(1, 0, 2)

    dp, bp = _degree(src_w, dst_w, HEW)
    dp = dp.reshape(NW, NN)
    bp = bp.reshape(NW, NE)

    ep = _phase(xs, src_t, dst_t)

    ef = pl.pallas_call(
        _combine_edges_body,
        out_shape=jax.ShapeDtypeStruct((NC, NE, HC), jnp.float32),
    )(ep, bp)

    npar = _phase(ef, dst_t, src_t)

    out = pl.pallas_call(
        _finish_body,
        out_shape=jax.ShapeDtypeStruct((NN, C), jnp.float32),
    )(npar, dp, W, b.reshape(1, C))
    return out


# speedup vs baseline: 22.8057x; 1.4850x over previous
import functools

import jax
import jax.numpy as jnp
from jax import lax
from jax.experimental import pallas as pl
from jax.experimental.pallas import tpu as pltpu
from jax.experimental.pallas import tpu_sc as plsc

NN = 10000
NE = 10000
NNZ = 320000
C = 128
HC = C // 2
NC = 2
NS = 16
NW = NC * NS
CH = 80
CHP = 125
EPT = NNZ // NS
NCHP = EPT // CHP
EPW = NNZ // NW
NCHD = EPW // CH
ACC_ROWS = 10240
ZR = ACC_ROWS // NS


def _zero_vec_loop(ref, rows, cols):
    z16 = jnp.zeros((16,), jnp.float32)
    g = cols // 16

    def body(i, carry):
        ref[i // g, pl.ds((i % g) * 16, 16)] = z16
        return carry

    lax.fori_loop(0, rows * g, body, 0)


def _phase_body(tab_hbm, gi_hbm, si_hbm, acc_out,
                gi_v, si_v, rows0_v, rows1_v, acc_sh, sem0, sem1):
    cid = lax.axis_index("c")
    sid = lax.axis_index("s")

    pltpu.sync_copy(gi_hbm.at[sid], gi_v)
    pltpu.sync_copy(si_hbm.at[sid], si_v)

    _zero_vec_loop(rows0_v, CHP, HC)
    zbase = pl.multiple_of(sid * ZR, 8)
    for k in range(ZR // CH):
        pltpu.sync_copy(rows0_v.at[pl.ds(0, CH)],
                        acc_sh.at[pl.ds(zbase + k * CH, CH)])
    plsc.subcore_barrier()

    tab = tab_hbm.at[cid]

    def gather(c, rows_v, sem):
        return pltpu.async_copy(tab.at[gi_v.at[c]], rows_v, sem)

    def gather_wait(c, rows_v, sem):
        pltpu.make_async_copy(tab.at[gi_v.at[c]], rows_v, sem).wait()

    def scatter(c, rows_v):
        pltpu.sync_copy(rows_v, acc_sh.at[si_v.at[c]], add=True)

    gather(0, rows0_v, sem0)

    def pair(i, carry):
        c0 = 2 * i
        gather_wait(c0, rows0_v, sem0)
        gather(c0 + 1, rows1_v, sem1)
        scatter(c0, rows0_v)
        gather_wait(c0 + 1, rows1_v, sem1)

        @pl.when(c0 + 2 < NCHP)
        def _():
            gather(c0 + 2, rows0_v, sem0)

        scatter(c0 + 1, rows1_v)
        return carry

    lax.fori_loop(0, NCHP // 2, pair, 0)
    plsc.subcore_barrier()

    pltpu.sync_copy(acc_sh.at[pl.ds(zbase, ZR)],
                    acc_out.at[cid, pl.ds(zbase, ZR)])


def _degree_body(gi_hbm, si_hbm, hew_hbm, dp_out, bp_out,
                 gi_v, si_v, hew_v, d_v, b_v):
    cid = lax.axis_index("c")
    sid = lax.axis_index("s")
    wid = sid * NC + cid

    pltpu.sync_copy(gi_hbm.at[wid], gi_v)
    pltpu.sync_copy(si_hbm.at[wid], si_v)
    pltpu.sync_copy(hew_hbm, hew_v)

    z16 = jnp.zeros((16,), jnp.float32)

    def zdb(i, carry):
        d_v[pl.ds(i * 16, 16)] = z16
        b_v[pl.ds(i * 16, 16)] = z16
        return carry

    lax.fori_loop(0, NN // 16, zdb, 0)

    ones16 = jnp.ones((16,), jnp.float32)

    def chunk(c, carry):
        for g in range(CH // 16):
            s16 = gi_v[c, pl.ds(g * 16, 16)]
            d16 = si_v[c, pl.ds(g * 16, 16)]
            w16 = plsc.load_gather(hew_v, [d16])
            plsc.addupdate_scatter(d_v, [s16], w16)
            plsc.addupdate_scatter(b_v, [d16], ones16)
        return carry

    lax.fori_loop(0, NCHD, chunk, 0)

    obase = pl.multiple_of(wid * NN, 8)
    pltpu.sync_copy(d_v, dp_out.at[pl.ds(obase, NN)])
    pltpu.sync_copy(b_v, bp_out.at[pl.ds(obase, NN)])


_MESH = plsc.VectorSubcoreMesh(core_axis_name="c", subcore_axis_name="s",
                               num_cores=NC, num_subcores=NS)
_SC_PARAMS = pltpu.CompilerParams(needs_layout_passes=False,
                                  use_tc_tiling_on_sc=False)

_degree = pl.kernel(
    _degree_body,
    out_type=(
        jax.ShapeDtypeStruct((NW * NN,), jnp.float32),
        jax.ShapeDtypeStruct((NW * NE,), jnp.float32),
    ),
    mesh=_MESH,
    compiler_params=_SC_PARAMS,
    scratch_types=(
        pltpu.VMEM((NCHD, CH), jnp.int32),
        pltpu.VMEM((NCHD, CH), jnp.int32),
        pltpu.VMEM((NE,), jnp.float32),
        pltpu.VMEM((NN,), jnp.float32),
        pltpu.VMEM((NE,), jnp.float32),
    ),
)

_phase = pl.kernel(
    _phase_body,
    out_type=jax.ShapeDtypeStruct((NC, ACC_ROWS, HC), jnp.float32),
    mesh=_MESH,
    compiler_params=_SC_PARAMS,
    scratch_types=(
        pltpu.VMEM((NCHP, CHP), jnp.int32),
        pltpu.VMEM((NCHP, CHP), jnp.int32),
        pltpu.VMEM((CHP, HC), jnp.float32),
        pltpu.VMEM((CHP, HC), jnp.float32),
        pltpu.VMEM_SHARED((ACC_ROWS, HC), jnp.float32),
        pltpu.SemaphoreType.DMA,
        pltpu.SemaphoreType.DMA,
    ),
)


def _combine_edges_body(ep_ref, bp_ref, out_ref):
    bsum = jnp.sum(bp_ref[...], axis=0)
    binv = jnp.where(bsum > 0, 1.0 / bsum, 0.0)
    out_ref[0] = ep_ref[0, :NE] * binv[:, None]
    out_ref[1] = ep_ref[1, :NE] * binv[:, None]


def _finish_body(np_ref, dp_ref, w_ref, b_ref, out_ref):
    dsum = jnp.sum(dp_ref[...], axis=0)
    dinv = jnp.where(dsum > 0, 1.0 / dsum, 0.0)
    t = jnp.concatenate([np_ref[0, :NN], np_ref[1, :NN]], axis=1)
    t = t * dinv[:, None]
    out_ref[...] = (jnp.dot(t, w_ref[...], preferred_element_type=jnp.float32)
                    + b_ref[...])


def kernel(x, HE, HEW, W, b):
    src = HE[0]
    dst = HE[1]
    src_w = src.reshape(NW, NCHD, CH)
    dst_w = dst.reshape(NW, NCHD, CH)
    src_t = src.reshape(NS, NCHP, CHP)
    dst_t = dst.reshape(NS, NCHP, CHP)
    xs = x.reshape(NN, NC, HC).transpose(1, 0, 2)

    dp, bp = _degree(src_w, dst_w, HEW)
    dp = dp.reshape(NW, NN)
    bp = bp.reshape(NW, NE)

    ep = _phase(xs, src_t, dst_t)

    ef = pl.pallas_call(
        _combine_edges_body,
        out_shape=jax.ShapeDtypeStruct((NC, NE, HC), jnp.float32),
    )(ep, bp)

    npar = _phase(ef, dst_t, src_t)

    out = pl.pallas_call(
        _finish_body,
        out_shape=jax.ShapeDtypeStruct((NN, C), jnp.float32),
    )(npar, dp, W, b.reshape(1, C))
    return out
